# baseline (speedup 1.0000x reference)
import dataclasses
import functools

import numpy as np
import jax
import jax.numpy as jnp
from jax import lax
from jax.experimental import pallas as pl
from jax.experimental.pallas import tpu as pltpu
from jax.experimental.pallas import tpu_sc as plsc

D_MODEL = 1024
D_FF = 4096
T = 2048
H = 16
DH = 64
NB = 32
CHUNK = 64
NCH = T // CHUNK

_INTERPRET = False


def _pos_enc():
    pos = np.arange(T)[:, None].astype(np.float32)
    i = np.arange(D_MODEL)[None, :].astype(np.float32)
    angle = pos / np.power(10000.0, (2.0 * np.floor(i / 2.0)) / D_MODEL)
    pe = np.zeros((T, D_MODEL), dtype=np.float32)
    pe[:, 0::2] = np.sin(angle[:, 0::2])
    pe[:, 1::2] = np.cos(angle[:, 1::2])
    return pe


_PE = _pos_enc()


def _sc_params():
    cp = pltpu.CompilerParams()
    if "needs_layout_passes" in pltpu.CompilerParams.__dataclass_fields__:
        cp = dataclasses.replace(cp, needs_layout_passes=False)
    return cp


def _sc_mesh():
    return plsc.VectorSubcoreMesh(core_axis_name="c", subcore_axis_name="s")


def _ring_gather(t_hbm, idx_v, o_hbm, base, bufs, gsems, ssems, per, ch):
    depth = len(bufs)
    nch = per // ch
    if nch == 1:
        pltpu.async_copy(t_hbm.at[idx_v], bufs[0], gsems[0]).wait()
        pltpu.sync_copy(bufs[0], o_hbm.at[pl.ds(base, per)])
        return
    for j in range(depth):
        pltpu.async_copy(t_hbm.at[idx_v.at[pl.ds(j * ch, ch)]],
                         bufs[j], gsems[j])
    for j in range(nch):
        b = j % depth
        pltpu.make_async_copy(t_hbm.at[idx_v.at[pl.ds(j * ch, ch)]],
                              bufs[b], gsems[b]).wait()
        pltpu.async_copy(bufs[b], o_hbm.at[pl.ds(base + j * ch, ch)],
                         ssems[b])
        nxt = j + depth
        pltpu.make_async_copy(
            bufs[b], o_hbm.at[pl.ds(base + j * ch, ch)], ssems[b]).wait()
        if nxt < nch:
            pltpu.async_copy(t_hbm.at[idx_v.at[pl.ds(nxt * ch, ch)]],
                             bufs[b], gsems[b])


def _sc_gather_rows(table, idx):
    M = idx.shape[0]
    Drow = table.shape[1]
    NW = 32
    per = M // NW
    ch = min(per, 128)

    nch = per // ch
    depth = min(nch, 4)

    @functools.partial(
        pl.kernel,
        out_type=jax.ShapeDtypeStruct((M, Drow), jnp.float32),
        mesh=_sc_mesh(),
        compiler_params=_sc_params(),
        scratch_types=[pltpu.VMEM((per,), jnp.int32)]
                      + [pltpu.VMEM((ch, Drow), jnp.float32)] * depth
                      + [pltpu.SemaphoreType.DMA] * (2 * depth),
    )
    def k(t_hbm, i_hbm, o_hbm, idx_v, *bufs_sems):
        bufs = bufs_sems[:depth]
        gsems = bufs_sems[depth:2 * depth]
        ssems = bufs_sems[2 * depth:]
        wid = lax.axis_index("s") * 2 + lax.axis_index("c")
        base = wid * per
        pltpu.sync_copy(i_hbm.at[pl.ds(base, per)], idx_v)
        _ring_gather(t_hbm, idx_v, o_hbm, base, bufs, gsems, ssems, per, ch)

    return k(table, idx)


def _sc_sort_gather(bflat, qkvflat):
    i32 = jnp.int32
    per, ch, depth = 1024, 128, 4

    @functools.partial(
        pl.kernel,
        out_type=(jax.ShapeDtypeStruct((H, T), i32),
                  jax.ShapeDtypeStruct((H, T), i32),
                  jax.ShapeDtypeStruct((H, T), i32),
                  jax.ShapeDtypeStruct((H, T), i32),
                  jax.ShapeDtypeStruct((H * T, 2 * DH), jnp.float32)),
        mesh=_sc_mesh(),
        compiler_params=_sc_params(),
        scratch_types=[pltpu.VMEM((T * H,), i32),
                       pltpu.VMEM((T,), i32),
                       pltpu.VMEM((T,), i32),
                       pltpu.VMEM((NB,), i32),
                       pltpu.VMEM((T,), i32),
                       pltpu.VMEM((T,), i32),
                       pltpu.VMEM((T,), i32),
                       pltpu.VMEM((T,), i32),
                       pltpu.SemaphoreType.DMA,
                       pltpu.VMEM((per,), i32)]
                      + [pltpu.VMEM((ch, 2 * DH), jnp.float32)] * depth
                      + [pltpu.SemaphoreType.DMA] * (2 * depth),
    )
    def k(b_hbm, qkv_hbm, tickg_hbm, spos_hbm, invg_hbm, sb_hbm, sqkv_hbm,
          ball_v, bh_v, rank_v, cnt_v, tick_v, sb_v, inv_v, tg_v, sem,
          idx_v, *bufs_sems):
        bufs = bufs_sems[:depth]
        gsems = bufs_sems[depth:2 * depth]
        ssems = bufs_sems[2 * depth:]
        core = lax.axis_index("c")
        sub = lax.axis_index("s")
        wid = sub * 2 + core

        @pl.when(wid < H)
        def _():
            h = wid
            pltpu.sync_copy(b_hbm, ball_v)
            lanes = lax.iota(i32, 16)
            cnt_v[pl.ds(0, 16)] = jnp.zeros((16,), i32)
            cnt_v[pl.ds(16, 16)] = jnp.zeros((16,), i32)

            @pl.loop(0, T, step=16)
            def _(t0):
                idx = (t0 + lanes) * H + h
                b = plsc.load_gather(ball_v, [idx])
                sc, last = plsc.scan_count(b)
                prior = plsc.load_gather(cnt_v, [b])
                plsc.store_scatter(cnt_v, [b], prior + sc, mask=last)
                bh_v[pl.ds(t0, 16)] = b
                rank_v[pl.ds(t0, 16)] = prior + sc - 1

            c0 = cnt_v[pl.ds(0, 16)]
            c1 = cnt_v[pl.ds(16, 16)]
            i0 = plsc.cumsum(c0)
            i1 = plsc.cumsum(c1)
            tot0 = jnp.sum(c0)
            cnt_v[pl.ds(0, 16)] = i0 - c0
            cnt_v[pl.ds(16, 16)] = i1 - c1 + tot0

            @pl.loop(0, T, step=16)
            def _(t0):
                b = bh_v[pl.ds(t0, 16)]
                s = plsc.load_gather(cnt_v, [b]) + rank_v[pl.ds(t0, 16)]
                tvec = t0 + lanes
                plsc.store_scatter(tick_v, [s], tvec)
                plsc.store_scatter(sb_v, [s], b)
                inv_v[pl.ds(t0, 16)] = s + h * T

            @pl.loop(0, T, step=16)
            def _(t0):
                tg_v[pl.ds(t0, 16)] = tick_v[pl.ds(t0, 16)] + h * T

            pltpu.sync_copy(tg_v, tickg_hbm.at[h])
            pltpu.sync_copy(tick_v, spos_hbm.at[h])
            pltpu.sync_copy(inv_v, invg_hbm.at[h])
            pltpu.sync_copy(sb_v, sb_hbm.at[h])

        plsc.subcore_barrier()
        gh = (sub // 2) * 2 + core
        half = sub % 2
        base = gh * T + half * per
        pltpu.sync_copy(tickg_hbm.at[gh, pl.ds(half * per, per)], idx_v)
        _ring_gather(qkv_hbm, idx_v, sqkv_hbm, base, bufs, gsems, ssems,
                     per, ch)

    return k(bflat, qkvflat)


def _tc_qkv_buckets(x_in, Wqk, Wv, rot, pe=None):
    BM = 256
    with_enc = pe is not None

    def body(*refs):
        if with_enc:
            rows_ref, pe_ref, wqk_ref, wv_ref, rot_ref, enc_ref, qkv3_ref, bkt_ref = refs
            x = rows_ref[...] * np.float32(np.sqrt(D_MODEL)) + pe_ref[...]
            enc_ref[...] = x
        else:
            x_ref, wqk_ref, wv_ref, rot_ref, qkv3_ref, bkt_ref = refs
            x = x_ref[...]
        qk = jnp.dot(x, wqk_ref[...], preferred_element_type=jnp.float32)
        v = jnp.dot(x, wv_ref[...], preferred_element_type=jnp.float32)
        iota32 = lax.broadcasted_iota(jnp.int32, (BM, NB), 1)
        cols = []
        for h in range(H):
            qh = qk[:, h * DH:(h + 1) * DH]
            qkv3_ref[h, :, :DH] = qh
            qkv3_ref[h, :, DH:] = v[:, h * DH:(h + 1) * DH]
            rx = jnp.dot(qh, rot_ref[...], preferred_element_type=jnp.float32)
            full = jnp.concatenate([rx, -rx], axis=1)
            m = jnp.max(full, axis=1, keepdims=True)
            cols.append(jnp.min(jnp.where(full == m, iota32, NB), axis=1,
                                keepdims=True))
        bkt_ref[...] = jnp.concatenate(cols, axis=1)

    x_spec = pl.BlockSpec((BM, D_MODEL), lambda i: (i, 0))
    w_spec = pl.BlockSpec((D_MODEL, D_MODEL), lambda i: (0, 0))
    rot_spec = pl.BlockSpec((DH, NB // 2), lambda i: (0, 0))
    qkv3_spec = pl.BlockSpec((H, BM, 2 * DH), lambda i: (0, i, 0))
    bkt_spec = pl.BlockSpec((BM, H), lambda i: (i, 0))
    out_shape = [jax.ShapeDtypeStruct((H, T, 2 * DH), jnp.float32),
                 jax.ShapeDtypeStruct((T, H), jnp.int32)]
    out_specs = [qkv3_spec, bkt_spec]
    in_specs = [x_spec, w_spec, w_spec, rot_spec]
    args = [x_in, Wqk, Wv, rot]
    if with_enc:
        in_specs = [x_spec, x_spec, w_spec, w_spec, rot_spec]
        args = [x_in, pe, Wqk, Wv, rot]
        out_shape = [jax.ShapeDtypeStruct((T, D_MODEL), jnp.float32)] + out_shape
        out_specs = [x_spec] + out_specs
    return pl.pallas_call(
        body, grid=(T // BM,), in_specs=in_specs, out_specs=out_specs,
        out_shape=out_shape, interpret=_INTERPRET,
    )(*args)


def _tc_attention(sqkv3, sb3, sp3, sbg3, spg3):
    def body(qv_ref, sb_ref, sp_ref, sbg_ref, spg_ref, o_ref):
        q = qv_ref[0, :, :DH]
        v = qv_ref[0, :, DH:]
        sbr = sb_ref[0]
        spr = sp_ref[0]
        nq = jnp.sqrt(jnp.sum(q * q, axis=1, keepdims=True))
        qn = q / (nq + np.float32(1e-6))
        qpn = jnp.concatenate([qn[T - CHUNK:], qn[:T - CHUNK]], axis=0)
        vp = jnp.concatenate([v[T - CHUNK:], v[:T - CHUNK]], axis=0)
        q3 = q.reshape(NCH, CHUNK, DH)
        qn3 = qn.reshape(NCH, CHUNK, DH)
        qpn3 = qpn.reshape(NCH, CHUNK, DH)
        v3 = v.reshape(NCH, CHUNK, DH)
        vp3 = vp.reshape(NCH, CHUNK, DH)
        k3 = jnp.concatenate([qpn3, qn3], axis=1)
        cv3 = jnp.concatenate([vp3, v3], axis=1)
        s3 = lax.dot_general(q3, k3, (((2,), (2,)), ((0,), (0,))),
                             preferred_element_type=jnp.float32)
        S = s3.reshape(T, 2 * CHUNK) / np.float32(np.sqrt(DH))
        sbga = sbg_ref[0]
        spga = spg_ref[0]
        sbp_g = jnp.concatenate([sbga[NCH - 1:], sbga[:NCH - 1]], axis=0)
        spp_g = jnp.concatenate([spga[NCH - 1:], spga[:NCH - 1]], axis=0)
        ska = jnp.concatenate([sbp_g, sbga], axis=1).astype(jnp.float32)
        spa = jnp.concatenate([spp_g, spga], axis=1).astype(jnp.float32)
        rowc = lax.broadcasted_iota(jnp.int32, (T, NCH), 0) // CHUNK
        rep = (rowc == lax.broadcasted_iota(jnp.int32, (T, NCH), 1))
        repf = rep.astype(jnp.float32)
        kb = jnp.dot(repf, ska, precision=lax.Precision.HIGHEST,
                     preferred_element_type=jnp.float32)
        kp = jnp.dot(repf, spa, precision=lax.Precision.HIGHEST,
                     preferred_element_type=jnp.float32)
        sbc = jnp.transpose(sbr).astype(jnp.float32)
        spc = jnp.transpose(spr).astype(jnp.float32)
        mask = (sbc == kb) & (spc >= kp)
        selfm = spc == kp
        S = jnp.where(mask, S, np.float32(-1e9))
        S = jnp.where(selfm, S - np.float32(1e5), S)
        m = jnp.max(S, axis=1, keepdims=True)
        e = jnp.exp(S - m)
        p = e / jnp.sum(e, axis=1, keepdims=True)
        p3 = p.reshape(NCH, CHUNK, 2 * CHUNK)
        o3 = lax.dot_general(p3, cv3, (((2,), (1,)), ((0,), (0,))),
                             preferred_element_type=jnp.float32)
        o_ref[0, :, :DH] = o3.reshape(T, DH)
        o_ref[0, :, DH:] = jnp.zeros((T, DH), jnp.float32)

    hd_spec = pl.BlockSpec((1, T, 2 * DH), lambda i: (i, 0, 0))
    id_spec = pl.BlockSpec((1, 1, T), lambda i: (i, 0, 0))
    g_spec = pl.BlockSpec((1, NCH, CHUNK), lambda i: (i, 0, 0))
    return pl.pallas_call(
        body, grid=(H,),
        in_specs=[hd_spec, id_spec, id_spec, g_spec, g_spec],
        out_specs=hd_spec,
        out_shape=jax.ShapeDtypeStruct((H, T, 2 * DH), jnp.float32),
        interpret=_INTERPRET,
    )(sqkv3, sb3, sp3, sbg3, spg3)


def _tc_out_ff(ou3, y1p, y2p, Wo, W1, b1, W2, b2):
    BM = 256

    def body(ou_ref, y1p_ref, y2p_ref, wo_ref, w1_ref, b1_ref, w2_ref, b2_ref,
             y1_ref, y2_ref):
        cat = jnp.concatenate([ou_ref[h, :, :DH] for h in range(H)], axis=1)
        y1 = y1p_ref[...] + jnp.dot(cat, wo_ref[...],
                                    preferred_element_type=jnp.float32)
        y1_ref[...] = y1
        h1 = jnp.maximum(jnp.dot(y1, w1_ref[...],
                                 preferred_element_type=jnp.float32)
                         + b1_ref[...], np.float32(0.0))
        y2_ref[...] = y2p_ref[...] + (jnp.dot(h1, w2_ref[...],
                                              preferred_element_type=jnp.float32)
                                      + b2_ref[...])

    x_spec = pl.BlockSpec((BM, D_MODEL), lambda i: (i, 0))
    return pl.pallas_call(
        body, grid=(T // BM,),
        in_specs=[pl.BlockSpec((H, BM, 2 * DH), lambda i: (0, i, 0)),
                  x_spec, x_spec,
                  pl.BlockSpec((D_MODEL, D_MODEL), lambda i: (0, 0)),
                  pl.BlockSpec((D_MODEL, D_FF), lambda i: (0, 0)),
                  pl.BlockSpec((1, D_FF), lambda i: (0, 0)),
                  pl.BlockSpec((D_FF, D_MODEL), lambda i: (0, 0)),
                  pl.BlockSpec((1, D_MODEL), lambda i: (0, 0))],
        out_specs=[x_spec, x_spec],
        out_shape=[jax.ShapeDtypeStruct((T, D_MODEL), jnp.float32),
                   jax.ShapeDtypeStruct((T, D_MODEL), jnp.float32)],
        interpret=_INTERPRET,
    )(ou3, y1p, y2p, Wo, W1, b1.reshape(1, D_FF), W2, b2.reshape(1, D_MODEL))


def _block(x, y1p, y2p, Wqk, Wv, Wo, W1, b1, W2, b2, rot, pe=None):
    if pe is not None:
        enc, qkv3, bkt = _tc_qkv_buckets(x, Wqk, Wv, rot, pe=pe)
        y1p = enc
        y2p = enc
    else:
        qkv3, bkt = _tc_qkv_buckets(x, Wqk, Wv, rot)
        enc = None
    tickg, spos, invg, sb, sqkv = _sc_sort_gather(
        bkt.reshape(T * H), qkv3.reshape(H * T, 2 * DH))
    del tickg
    os3 = _tc_attention(sqkv.reshape(H, T, 2 * DH),
                        sb.reshape(H, 1, T), spos.reshape(H, 1, T),
                        sb.reshape(H, NCH, CHUNK), spos.reshape(H, NCH, CHUNK))
    ou = _sc_gather_rows(os3.reshape(H * T, 2 * DH), invg.reshape(H * T))
    y1, y2 = _tc_out_ff(ou.reshape(H, T, 2 * DH), y1p, y2p, Wo, W1, b1, W2, b2)
    return enc, y1, y2


def kernel(xs, emb, Wqk0, Wv0, Wo0, W10, b10, W20, b20,
           Wqk1, Wv1, Wo1, W11, b11, W21, b21):
    xsf = xs.reshape(T)
    rot = jax.random.normal(jax.random.key(42), (DH, NB // 2), dtype=jnp.float32)
    pe = jnp.asarray(_PE)
    rows = _sc_gather_rows(emb, xsf)
    enc, y1, y2 = _block(rows, None, None, Wqk0, Wv0, Wo0, W10, b10, W20, b20,
                         rot, pe=pe)
    _, y1, y2 = _block(y2, y1, y2, Wqk1, Wv1, Wo1, W11, b11, W21, b21, rot)
    return (enc.reshape(1, T, D_MODEL), y1.reshape(1, T, D_MODEL),
            y2.reshape(1, T, D_MODEL))


# speedup vs baseline: 1.0426x; 1.0426x over previous
import dataclasses
import functools

import numpy as np
import jax
import jax.numpy as jnp
from jax import lax
from jax.experimental import pallas as pl
from jax.experimental.pallas import tpu as pltpu
from jax.experimental.pallas import tpu_sc as plsc

D_MODEL = 1024
D_FF = 4096
T = 2048
H = 16
DH = 64
NB = 32
CHUNK = 64
NCH = T // CHUNK

_INTERPRET = False


def _pos_enc():
    pos = np.arange(T)[:, None].astype(np.float32)
    i = np.arange(D_MODEL)[None, :].astype(np.float32)
    angle = pos / np.power(10000.0, (2.0 * np.floor(i / 2.0)) / D_MODEL)
    pe = np.zeros((T, D_MODEL), dtype=np.float32)
    pe[:, 0::2] = np.sin(angle[:, 0::2])
    pe[:, 1::2] = np.cos(angle[:, 1::2])
    return pe


_PE = _pos_enc()


def _sc_params():
    cp = pltpu.CompilerParams()
    if "needs_layout_passes" in pltpu.CompilerParams.__dataclass_fields__:
        cp = dataclasses.replace(cp, needs_layout_passes=False)
    return cp


def _sc_mesh():
    return plsc.VectorSubcoreMesh(core_axis_name="c", subcore_axis_name="s")


def _ring_gather(t_hbm, idx_v, o_hbm, base, bufs, gsems, ssems, per, ch):
    depth = len(bufs)
    nch = per // ch
    if nch == 1:
        pltpu.async_copy(t_hbm.at[idx_v], bufs[0], gsems[0]).wait()
        pltpu.sync_copy(bufs[0], o_hbm.at[pl.ds(base, per)])
        return
    for j in range(depth):
        pltpu.async_copy(t_hbm.at[idx_v.at[pl.ds(j * ch, ch)]],
                         bufs[j], gsems[j])
    for j in range(nch):
        b = j % depth
        pltpu.make_async_copy(t_hbm.at[idx_v.at[pl.ds(j * ch, ch)]],
                              bufs[b], gsems[b]).wait()
        pltpu.async_copy(bufs[b], o_hbm.at[pl.ds(base + j * ch, ch)],
                         ssems[b])
        nxt = j + depth
        pltpu.make_async_copy(
            bufs[b], o_hbm.at[pl.ds(base + j * ch, ch)], ssems[b]).wait()
        if nxt < nch:
            pltpu.async_copy(t_hbm.at[idx_v.at[pl.ds(nxt * ch, ch)]],
                             bufs[b], gsems[b])


def _sc_gather_rows(table, idx):
    M = idx.shape[0]
    Drow = table.shape[1]
    NW = 32
    per = M // NW
    ch = min(per, 128)

    nch = per // ch
    depth = min(nch, 4)

    @functools.partial(
        pl.kernel,
        out_type=jax.ShapeDtypeStruct((M, Drow), jnp.float32),
        mesh=_sc_mesh(),
        compiler_params=_sc_params(),
        scratch_types=[pltpu.VMEM((per,), jnp.int32)]
                      + [pltpu.VMEM((ch, Drow), jnp.float32)] * depth
                      + [pltpu.SemaphoreType.DMA] * (2 * depth),
    )
    def k(t_hbm, i_hbm, o_hbm, idx_v, *bufs_sems):
        bufs = bufs_sems[:depth]
        gsems = bufs_sems[depth:2 * depth]
        ssems = bufs_sems[2 * depth:]
        wid = lax.axis_index("s") * 2 + lax.axis_index("c")
        base = wid * per
        pltpu.sync_copy(i_hbm.at[pl.ds(base, per)], idx_v)
        _ring_gather(t_hbm, idx_v, o_hbm, base, bufs, gsems, ssems, per, ch)

    return k(table, idx)


def _sc_sort(bflat):
    i32 = jnp.int32

    @functools.partial(
        pl.kernel,
        out_type=(jax.ShapeDtypeStruct((H, T), i32),
                  jax.ShapeDtypeStruct((H, T), i32),
                  jax.ShapeDtypeStruct((H, T), i32),
                  jax.ShapeDtypeStruct((H, T), i32)),
        mesh=_sc_mesh(),
        compiler_params=_sc_params(),
        scratch_types=[pltpu.VMEM((T * H,), i32),
                       pltpu.VMEM((T,), i32),
                       pltpu.VMEM((T,), i32),
                       pltpu.VMEM((NB,), i32),
                       pltpu.VMEM((T,), i32),
                       pltpu.VMEM((T,), i32),
                       pltpu.VMEM((T,), i32),
                       pltpu.VMEM((T,), i32),
                       pltpu.SemaphoreType.DMA],
    )
    def k(b_hbm, tickg_hbm, spos_hbm, invg_hbm, sb_hbm,
          ball_v, bh_v, rank_v, cnt_v, tick_v, sb_v, inv_v, tg_v, sem):
        wid = lax.axis_index("s") * 2 + lax.axis_index("c")

        @pl.when(wid < H)
        def _():
            h = wid
            pltpu.sync_copy(b_hbm, ball_v)
            lanes = lax.iota(i32, 16)
            cnt_v[pl.ds(0, 16)] = jnp.zeros((16,), i32)
            cnt_v[pl.ds(16, 16)] = jnp.zeros((16,), i32)

            @pl.loop(0, T, step=16)
            def _(t0):
                idx = (t0 + lanes) * H + h
                b = plsc.load_gather(ball_v, [idx])
                sc, last = plsc.scan_count(b)
                prior = plsc.load_gather(cnt_v, [b])
                plsc.store_scatter(cnt_v, [b], prior + sc, mask=last)
                bh_v[pl.ds(t0, 16)] = b
                rank_v[pl.ds(t0, 16)] = prior + sc - 1

            c0 = cnt_v[pl.ds(0, 16)]
            c1 = cnt_v[pl.ds(16, 16)]
            i0 = plsc.cumsum(c0)
            i1 = plsc.cumsum(c1)
            tot0 = jnp.sum(c0)
            cnt_v[pl.ds(0, 16)] = i0 - c0
            cnt_v[pl.ds(16, 16)] = i1 - c1 + tot0

            @pl.loop(0, T, step=16)
            def _(t0):
                b = bh_v[pl.ds(t0, 16)]
                s = plsc.load_gather(cnt_v, [b]) + rank_v[pl.ds(t0, 16)]
                tvec = t0 + lanes
                plsc.store_scatter(tick_v, [s], tvec)
                plsc.store_scatter(sb_v, [s], b)
                inv_v[pl.ds(t0, 16)] = s + h * T

            @pl.loop(0, T, step=16)
            def _(t0):
                tg_v[pl.ds(t0, 16)] = tick_v[pl.ds(t0, 16)] + h * T

            pltpu.sync_copy(tg_v, tickg_hbm.at[h])
            pltpu.sync_copy(tick_v, spos_hbm.at[h])
            pltpu.sync_copy(inv_v, invg_hbm.at[h])
            pltpu.sync_copy(sb_v, sb_hbm.at[h])

    return k(bflat)


def _tc_qkv_buckets(rows, pe, Wqk, Wv, rot):
    BM = 256

    def body(rows_ref, pe_ref, wqk_ref, wv_ref, rot_ref,
             enc_ref, qkv3_ref, bkt_ref):
        x = rows_ref[...] * np.float32(np.sqrt(D_MODEL)) + pe_ref[...]
        enc_ref[...] = x
        _qkv_buckets_compute(x, wqk_ref[...], wv_ref[...], rot_ref[...],
                             qkv3_ref, bkt_ref, BM)

    x_spec = pl.BlockSpec((BM, D_MODEL), lambda i: (i, 0))
    w_spec = pl.BlockSpec((D_MODEL, D_MODEL), lambda i: (0, 0))
    return pl.pallas_call(
        body, grid=(T // BM,),
        in_specs=[x_spec, x_spec, w_spec, w_spec,
                  pl.BlockSpec((DH, NB // 2), lambda i: (0, 0))],
        out_specs=[x_spec,
                   pl.BlockSpec((H, BM, 2 * DH), lambda i: (0, i, 0)),
                   pl.BlockSpec((BM, H), lambda i: (i, 0))],
        out_shape=[jax.ShapeDtypeStruct((T, D_MODEL), jnp.float32),
                   jax.ShapeDtypeStruct((H, T, 2 * DH), jnp.float32),
                   jax.ShapeDtypeStruct((T, H), jnp.int32)],
        interpret=_INTERPRET,
    )(rows, pe, Wqk, Wv, rot)


def _tc_attention(sqkv3, sb3, sp3, sbg3, spg3):
    def body(qv_ref, sb_ref, sp_ref, sbg_ref, spg_ref, o_ref):
        q = qv_ref[0, :, :DH]
        v = qv_ref[0, :, DH:]
        sbr = sb_ref[0]
        spr = sp_ref[0]
        nq = jnp.sqrt(jnp.sum(q * q, axis=1, keepdims=True))
        qn = q / (nq + np.float32(1e-6))
        qpn = jnp.concatenate([qn[T - CHUNK:], qn[:T - CHUNK]], axis=0)
        vp = jnp.concatenate([v[T - CHUNK:], v[:T - CHUNK]], axis=0)
        q3 = q.reshape(NCH, CHUNK, DH)
        qn3 = qn.reshape(NCH, CHUNK, DH)
        qpn3 = qpn.reshape(NCH, CHUNK, DH)
        v3 = v.reshape(NCH, CHUNK, DH)
        vp3 = vp.reshape(NCH, CHUNK, DH)
        k3 = jnp.concatenate([qpn3, qn3], axis=1)
        cv3 = jnp.concatenate([vp3, v3], axis=1)
        s3 = lax.dot_general(q3, k3, (((2,), (2,)), ((0,), (0,))),
                             preferred_element_type=jnp.float32)
        S = s3.reshape(T, 2 * CHUNK) / np.float32(np.sqrt(DH))
        sbga = sbg_ref[0]
        spga = spg_ref[0]
        sbp_g = jnp.concatenate([sbga[NCH - 1:], sbga[:NCH - 1]], axis=0)
        spp_g = jnp.concatenate([spga[NCH - 1:], spga[:NCH - 1]], axis=0)
        ska = jnp.concatenate([sbp_g, sbga], axis=1).astype(jnp.float32)
        spa = jnp.concatenate([spp_g, spga], axis=1).astype(jnp.float32)
        rowc = lax.broadcasted_iota(jnp.int32, (T, NCH), 0) // CHUNK
        rep = (rowc == lax.broadcasted_iota(jnp.int32, (T, NCH), 1))
        repf = rep.astype(jnp.float32)
        kb = jnp.dot(repf, ska, precision=lax.Precision.HIGHEST,
                     preferred_element_type=jnp.float32)
        kp = jnp.dot(repf, spa, precision=lax.Precision.HIGHEST,
                     preferred_element_type=jnp.float32)
        sbc = jnp.transpose(sbr).astype(jnp.float32)
        spc = jnp.transpose(spr).astype(jnp.float32)
        mask = (sbc == kb) & (spc >= kp)
        selfm = spc == kp
        S = jnp.where(mask, S, np.float32(-1e9))
        S = jnp.where(selfm, S - np.float32(1e5), S)
        m = jnp.max(S, axis=1, keepdims=True)
        e = jnp.exp(S - m)
        p = e / jnp.sum(e, axis=1, keepdims=True)
        p3 = p.reshape(NCH, CHUNK, 2 * CHUNK)
        o3 = lax.dot_general(p3, cv3, (((2,), (1,)), ((0,), (0,))),
                             preferred_element_type=jnp.float32)
        o_ref[0, :, :DH] = o3.reshape(T, DH)
        o_ref[0, :, DH:] = jnp.zeros((T, DH), jnp.float32)

    hd_spec = pl.BlockSpec((1, T, 2 * DH), lambda i: (i, 0, 0))
    id_spec = pl.BlockSpec((1, 1, T), lambda i: (i, 0, 0))
    g_spec = pl.BlockSpec((1, NCH, CHUNK), lambda i: (i, 0, 0))
    return pl.pallas_call(
        body, grid=(H,),
        in_specs=[hd_spec, id_spec, id_spec, g_spec, g_spec],
        out_specs=hd_spec,
        out_shape=jax.ShapeDtypeStruct((H, T, 2 * DH), jnp.float32),
        interpret=_INTERPRET,
    )(sqkv3, sb3, sp3, sbg3, spg3)


def _qkv_buckets_compute(x, wqk, wv, rot, qkv3_ref, bkt_ref, BM):
    qk = jnp.dot(x, wqk, preferred_element_type=jnp.float32)
    v = jnp.dot(x, wv, preferred_element_type=jnp.float32)
    iota32 = lax.broadcasted_iota(jnp.int32, (BM, NB), 1)
    cols = []
    for h in range(H):
        qh = qk[:, h * DH:(h + 1) * DH]
        qkv3_ref[h, :, :DH] = qh
        qkv3_ref[h, :, DH:] = v[:, h * DH:(h + 1) * DH]
        rx = jnp.dot(qh, rot, preferred_element_type=jnp.float32)
        full = jnp.concatenate([rx, -rx], axis=1)
        m = jnp.max(full, axis=1, keepdims=True)
        cols.append(jnp.min(jnp.where(full == m, iota32, NB), axis=1,
                            keepdims=True))
    bkt_ref[...] = jnp.concatenate(cols, axis=1)


def _tc_out_ff(ou3, y1p, y2p, Wo, W1, b1, W2, b2, nxt=None):
    BM = 128 if nxt is not None else 256

    def body(*refs):
        if nxt is None:
            (ou_ref, y1p_ref, y2p_ref, wo_ref, w1_ref, b1_ref, w2_ref, b2_ref,
             y1_ref, y2_ref) = refs
        else:
            (ou_ref, y1p_ref, y2p_ref, wo_ref, w1_ref, b1_ref, w2_ref, b2_ref,
             wqkn_ref, wvn_ref, rot_ref,
             y1_ref, y2_ref, qkv3_ref, bkt_ref) = refs
        cat = jnp.concatenate([ou_ref[h, :, :DH] for h in range(H)], axis=1)
        y1 = y1p_ref[...] + jnp.dot(cat, wo_ref[...],
                                    preferred_element_type=jnp.float32)
        y1_ref[...] = y1
        h1 = jnp.maximum(jnp.dot(y1, w1_ref[...],
                                 preferred_element_type=jnp.float32)
                         + b1_ref[...], np.float32(0.0))
        y2 = y2p_ref[...] + (jnp.dot(h1, w2_ref[...],
                                     preferred_element_type=jnp.float32)
                             + b2_ref[...])
        y2_ref[...] = y2
        if nxt is not None:
            _qkv_buckets_compute(y2, wqkn_ref[...], wvn_ref[...], rot_ref[...],
                                 qkv3_ref, bkt_ref, BM)

    x_spec = pl.BlockSpec((BM, D_MODEL), lambda i: (i, 0))
    w_spec = pl.BlockSpec((D_MODEL, D_MODEL), lambda i: (0, 0))
    in_specs = [pl.BlockSpec((H, BM, 2 * DH), lambda i: (0, i, 0)),
                x_spec, x_spec, w_spec,
                pl.BlockSpec((D_MODEL, D_FF), lambda i: (0, 0)),
                pl.BlockSpec((1, D_FF), lambda i: (0, 0)),
                pl.BlockSpec((D_FF, D_MODEL), lambda i: (0, 0)),
                pl.BlockSpec((1, D_MODEL), lambda i: (0, 0))]
    out_specs = [x_spec, x_spec]
    out_shape = [jax.ShapeDtypeStruct((T, D_MODEL), jnp.float32),
                 jax.ShapeDtypeStruct((T, D_MODEL), jnp.float32)]
    args = [ou3, y1p, y2p, Wo, W1, b1.reshape(1, D_FF), W2,
            b2.reshape(1, D_MODEL)]
    if nxt is not None:
        Wqk_n, Wv_n, rot = nxt
        in_specs += [w_spec, w_spec,
                     pl.BlockSpec((DH, NB // 2), lambda i: (0, 0))]
        out_specs += [pl.BlockSpec((H, BM, 2 * DH), lambda i: (0, i, 0)),
                      pl.BlockSpec((BM, H), lambda i: (i, 0))]
        out_shape += [jax.ShapeDtypeStruct((H, T, 2 * DH), jnp.float32),
                      jax.ShapeDtypeStruct((T, H), jnp.int32)]
        args += [Wqk_n, Wv_n, rot]
    return pl.pallas_call(
        body, grid=(T // BM,), in_specs=in_specs, out_specs=out_specs,
        out_shape=out_shape, interpret=_INTERPRET,
    )(*args)


def _block(qkv3, bkt, y1p, y2p, Wo, W1, b1, W2, b2, nxt=None):
    tickg, spos, invg, sb = _sc_sort(bkt.reshape(T * H))
    sqkv = _sc_gather_rows(qkv3.reshape(H * T, 2 * DH), tickg.reshape(H * T))
    os3 = _tc_attention(sqkv.reshape(H, T, 2 * DH),
                        sb.reshape(H, 1, T), spos.reshape(H, 1, T),
                        sb.reshape(H, NCH, CHUNK), spos.reshape(H, NCH, CHUNK))
    ou = _sc_gather_rows(os3.reshape(H * T, 2 * DH), invg.reshape(H * T))
    return _tc_out_ff(ou.reshape(H, T, 2 * DH), y1p, y2p, Wo, W1, b1, W2, b2,
                      nxt=nxt)


def kernel(xs, emb, Wqk0, Wv0, Wo0, W10, b10, W20, b20,
           Wqk1, Wv1, Wo1, W11, b11, W21, b21):
    xsf = xs.reshape(T)
    rot = jax.random.normal(jax.random.key(42), (DH, NB // 2), dtype=jnp.float32)
    pe = jnp.asarray(_PE)
    rows = _sc_gather_rows(emb, xsf)
    enc, qkv0, bkt0 = _tc_qkv_buckets(rows, pe, Wqk0, Wv0, rot)
    y1a, y2a, qkv1, bkt1 = _block(qkv0, bkt0, enc, enc, Wo0, W10, b10,
                                  W20, b20, nxt=(Wqk1, Wv1, rot))
    y1, y2 = _block(qkv1, bkt1, y1a, y2a, Wo1, W11, b11, W21, b21)
    return (enc.reshape(1, T, D_MODEL), y1.reshape(1, T, D_MODEL),
            y2.reshape(1, T, D_MODEL))
